# Initial kernel scaffold; baseline (speedup 1.0000x reference)
import jax
import jax.numpy as jnp
from jax.experimental import pallas as pl


def kernel(x, embedding_weight):
    raise NotImplementedError("write your pallas kernel here")


# speedup vs baseline: 3.4655x; 3.4655x over previous
import functools

import jax
import jax.numpy as jnp
from jax import lax
from jax.experimental import pallas as pl
from jax.experimental.pallas import tpu as pltpu
from jax.experimental.pallas import tpu_sc as plsc

VOCAB = 1000
EMB = 64
NC = 2
NS = 16
NW = NC * NS
CHUNK = 128
NBUF = 4


def _relu_body(w_ref, o_ref):
    o_ref[...] = jnp.maximum(w_ref[...], 0.0)


def _relu_table(w):
    return pl.pallas_call(
        _relu_body,
        out_shape=jax.ShapeDtypeStruct(w.shape, w.dtype),
    )(w)


def _make_gather(batch):
    per_w = batch // NW
    n_chunks = per_w // CHUNK
    n_groups = n_chunks // NBUF
    assert per_w % CHUNK == 0 and n_chunks % NBUF == 0

    mesh = plsc.VectorSubcoreMesh(core_axis_name="c", subcore_axis_name="s")

    @functools.partial(
        pl.kernel,
        mesh=mesh,
        compiler_params=pltpu.CompilerParams(use_tc_tiling_on_sc=False),
        out_type=jax.ShapeDtypeStruct((batch, EMB), jnp.float32),
        scratch_types=(
            [pltpu.VMEM((per_w,), jnp.int32)]
            + [pltpu.VMEM((CHUNK, EMB), jnp.float32) for _ in range(NBUF)]
            + [pltpu.SemaphoreType.DMA for _ in range(2 * NBUF)]
        ),
    )
    def gather_kernel(idx_hbm, table_hbm, out_hbm, idx_v, *bufs_sems):
        bufs = bufs_sems[:NBUF]
        gsem = bufs_sems[NBUF:2 * NBUF]
        osem = bufs_sems[2 * NBUF:]

        wid = lax.axis_index("s") * NC + lax.axis_index("c")
        base = wid * per_w

        pltpu.sync_copy(idx_hbm.at[pl.ds(base, per_w)], idx_v)

        def start_gather(j, b):
            idx_slice = idx_v.at[pl.ds(j * CHUNK, CHUNK)]
            pltpu.make_async_copy(
                table_hbm.at[idx_slice], bufs[b], gsem[b]).start()

        def start_write(j, b):
            pltpu.make_async_copy(
                bufs[b], out_hbm.at[pl.ds(base + j * CHUNK, CHUNK)],
                osem[b]).start()

        def wait_gather(b):
            pltpu.make_async_copy(
                table_hbm.at[idx_v.at[pl.ds(0, CHUNK)]], bufs[b],
                gsem[b]).wait()

        def wait_write(b):
            pltpu.make_async_copy(
                bufs[b], out_hbm.at[pl.ds(base, CHUNK)], osem[b]).wait()

        for b in range(NBUF):
            start_gather(b, b)

        def group_body(g, _):
            jj = g * NBUF
            for b in range(NBUF):
                wait_gather(b)
                start_write(jj + b, b)
            for b in range(NBUF):
                nj = jj + NBUF + b

                @pl.when(nj < n_chunks)
                def _():
                    wait_write(b)
                    start_gather(nj, b)

            return 0

        lax.fori_loop(0, n_groups, group_body, 0)

        for b in range(NBUF):
            wait_write(b)

    return gather_kernel


def kernel(x, embedding_weight):
    batch = x.shape[0] * x.shape[1]
    table = _relu_table(embedding_weight)
    out = _make_gather(batch)(x.reshape(-1), table)
    return out.reshape(x.shape[0], x.shape[1], EMB)


# speedup vs baseline: 6.9289x; 1.9994x over previous
import functools

import jax
import jax.numpy as jnp
from jax import lax
from jax.experimental import pallas as pl
from jax.experimental.pallas import tpu as pltpu
from jax.experimental.pallas import tpu_sc as plsc

EMB = 64
NC = 2
NS = 16
NW = NC * NS
BL = 128
NBUF = 4


def _relu_body(w_ref, o_ref):
    o_ref[...] = jnp.maximum(w_ref[...], 0.0)


def _relu_table(w):
    return pl.pallas_call(
        _relu_body,
        out_shape=jax.ShapeDtypeStruct(w.shape, w.dtype),
    )(w)


def _make_gather(nb, nh):
    nbt = nb // BL
    assert nbt == NW and nh % NBUF == 0
    mesh = plsc.VectorSubcoreMesh(core_axis_name="c", subcore_axis_name="s")

    @functools.partial(
        pl.kernel,
        mesh=mesh,
        compiler_params=pltpu.CompilerParams(
            use_tc_tiling_on_sc=False, needs_layout_passes=False),
        out_type=jax.ShapeDtypeStruct((nh, 8, nbt, 8, BL), jnp.float32),
        scratch_types=(
            [pltpu.VMEM((nh, BL), jnp.int32)]
            + [pltpu.VMEM((BL, EMB), jnp.float32) for _ in range(NBUF)]
            + [pltpu.VMEM((8, 8, 129), jnp.float32) for _ in range(NBUF)]
            + [pltpu.SemaphoreType.DMA for _ in range(2 * NBUF)]
        ),
    )
    def gather_kernel(xt_hbm, table_hbm, out_hbm, idx_v, *bufs):
        rows = bufs[:NBUF]
        tb = bufs[NBUF:2 * NBUF]
        gsem = bufs[2 * NBUF:3 * NBUF]
        osem = bufs[3 * NBUF:]

        wid = lax.axis_index("s") * NC + lax.axis_index("c")

        pltpu.sync_copy(xt_hbm.at[:, pl.ds(wid * BL, BL)], idx_v)

        def start_gather(h, s):
            pltpu.make_async_copy(
                table_hbm.at[idx_v.at[h]], rows[s], gsem[s]).start()

        def wait_gather(s):
            pltpu.make_async_copy(
                table_hbm.at[idx_v.at[0]], rows[s], gsem[s]).wait()

        def start_write(h, s):
            pltpu.make_async_copy(
                tb[s].at[:, :, pl.ds(0, BL)], out_hbm.at[h, :, wid],
                osem[s]).start()

        def wait_write(s):
            pltpu.make_async_copy(
                tb[s].at[:, :, pl.ds(0, BL)], out_hbm.at[0, :, wid],
                osem[s]).wait()

        def transpose(s):
            for ec in range(EMB // 16):
                e0 = ec * 16
                ev = lax.broadcasted_iota(jnp.int32, (16,), 0) + e0
                etv = lax.shift_right_logical(ev, 3)
                esv = ev & 7

                def bl_body(i, _, e0=e0, etv=etv, esv=esv):
                    for u in range(8):
                        bl = i * 8 + u
                        v = rows[s][bl, pl.ds(e0, 16)]
                        blv = jnp.full((16,), 0, jnp.int32) + bl
                        plsc.store_scatter(tb[s], [etv, esv, blv], v)
                    return 0

                lax.fori_loop(0, BL // 8, bl_body, 0)

        for s in range(NBUF):
            start_gather(s, s)

        def group_body(hh, _):
            for s in range(NBUF):
                h = hh * NBUF + s
                wait_gather(s)

                @pl.when(hh >= 1)
                def _():
                    wait_write(s)

                transpose(s)
                start_write(h, s)

                @pl.when(hh < nh // NBUF - 1)
                def _():
                    start_gather(h + NBUF, s)

            return 0

        lax.fori_loop(0, nh // NBUF, group_body, 0)
        for s in range(NBUF):
            wait_write(s)

    return gather_kernel


def kernel(x, embedding_weight):
    nb, nh = x.shape
    table = _relu_table(embedding_weight)
    xt = x.T
    y = _make_gather(nb, nh)(xt, table)
    return y.transpose(2, 4, 0, 1, 3).reshape(nb, nh, EMB)
